# baseline (speedup 1.0000x reference)
import functools

import jax
import jax.numpy as jnp
from jax import lax
from jax.experimental import pallas as pl
from jax.experimental.pallas import tpu as pltpu
from jax.experimental.pallas import tpu_sc as plsc

N = 10000
E = 320000
D = 128
ZD = 64
PE = 8
T = 1024

NC = 2
NS = 16
HALF = N // NC
C = 128
EPT = 20480
TBL = 10016
HN = 5120
RPT = HN // NS
CR = 40


def _sc_aggregate_body(edge_hbm, q_hbm, tn_hbm, zf_hbm, zi_hbm,
                       out_q, out_cnt,
                       msrc, mdst, table, q_buf, cnt_loc, iidx, sidx2,
                       gsem, ssem, acc_q, acc_cnt):
  c_idx = lax.axis_index("c")
  s_idx = lax.axis_index("s")
  lo = c_idx * HALF

  onei32 = jnp.ones((16,), jnp.int32)
  onef32 = jnp.ones((16,), jnp.float32)
  iota16 = lax.iota(jnp.int32, 16)

  iidx[0, pl.ds(0, 16)] = iota16
  iidx[0, pl.ds(16, 16)] = iota16 + 16
  iidx[0, pl.ds(24, 16)] = iota16 + 24

  slab_src = pltpu.make_async_copy(edge_hbm.at[0, s_idx],
                                   msrc.at[pl.ds(0, EPT)], gsem.at[0])
  slab_dst = pltpu.make_async_copy(edge_hbm.at[1, s_idx],
                                   mdst.at[pl.ds(0, EPT)], gsem.at[1])
  slab_src.start()
  slab_dst.start()

  base = s_idx * RPT
  pltpu.sync_copy(
      [zf_hbm, zf_hbm.at[pl.ds(0, CR)], zi_hbm, tn_hbm],
      [acc_q.at[pl.ds(base, RPT)], cnt_loc, table,
       q_buf.at[0, pl.ds(0, T // C)]])

  @pl.when(s_idx == 0)
  def _():
    pltpu.sync_copy(zf_hbm.at[pl.ds(0, CR)], acc_cnt)

  for i in range(T // 16):
    t16 = plsc.bitcast(
        q_buf[0, i // (C // 16), pl.ds((i % (C // 16)) * 16, 16)], jnp.int32)
    plsc.store_scatter(table, [t16], onei32)

  slab_src.wait()
  slab_dst.wait()

  neg1 = jnp.full((16,), -1, jnp.int32)

  def compact_one(cnt, sv, dv):
    t = plsc.load_gather(table, [jnp.maximum(sv, 0)])
    m = (t == 1) & (sv >= lo) & (sv < lo + HALF)
    rebased = jnp.where(m, sv - lo, 0)
    plsc.addupdate_scatter(
        cnt_loc,
        [lax.shift_right_logical(rebased, 7),
         jnp.bitwise_and(rebased, 127)],
        onef32, mask=m)
    plsc.store_compressed(msrc.at[pl.ds(cnt, 16)], rebased, mask=m)
    plsc.store_compressed(mdst.at[pl.ds(cnt, 16)], dv, mask=m)
    return cnt + jnp.sum(m.astype(jnp.int32))

  def compact_vec(v, cnt):
    off = v * 32
    sv0 = msrc[pl.ds(off, 16)]
    dv0 = mdst[pl.ds(off, 16)]
    sv1 = msrc[pl.ds(off + 16, 16)]
    dv1 = mdst[pl.ds(off + 16, 16)]
    cnt = compact_one(cnt, sv0, dv0)
    return compact_one(cnt, sv1, dv1)
  cnt = lax.fori_loop(0, EPT // 32, compact_vec, jnp.int32(0))

  for k in range(C // 16):
    msrc[pl.ds(cnt + k * 16, 16)] = neg1
    mdst[pl.ds(cnt + k * 16, 16)] = neg1
  nch = lax.div(cnt + C - 1, jnp.int32(C))

  plsc.subcore_barrier()

  def g_desc(ci, b):
    return pltpu.make_async_copy(
        q_hbm.at[plsc.Indices(mdst.at[pl.ds(ci * C, C)], ignored_value=-1)],
        q_buf.at[b], gsem.at[b])

  def s_desc(b):
    return pltpu.make_async_copy(
        q_buf.at[b],
        acc_q.at[plsc.Indices(sidx2.at[b], ignored_value=-1)],
        ssem.at[b])

  @pl.when(nch > 0)
  def _():
    g_desc(0, 0).start()

  def chunk(ci, _):
    b = lax.rem(ci, 2)
    nb = 1 - b

    @pl.when(ci >= 1)
    def _():
      s_desc(nb).wait()

    @pl.when(ci + 1 < nch)
    def _():
      g_desc(ci + 1, nb).start()

    g_desc(ci, b).wait()
    for j in range(C // 16):
      sidx2[b, pl.ds(j * 16, 16)] = msrc[pl.ds(ci * C + j * 16, 16)]
    s_desc(b).start(add=True)
    return 0
  lax.fori_loop(0, nch, chunk, 0)

  @pl.when(nch > 0)
  def _():
    s_desc(lax.rem(nch - 1, 2)).wait()

  pltpu.sync_copy(cnt_loc, acc_cnt.at[plsc.Indices(iidx.at[0])], add=True)

  plsc.subcore_barrier()

  pltpu.sync_copy(acc_q.at[pl.ds(base, RPT)],
                  out_q.at[c_idx, pl.ds(base, RPT)])

  @pl.when(s_idx == 0)
  def _():
    pltpu.sync_copy(acc_cnt, out_cnt.at[c_idx])


_sc_aggregate = functools.partial(
    pl.kernel,
    out_type=(jax.ShapeDtypeStruct((NC, HN, D), jnp.float32),
              jax.ShapeDtypeStruct((NC, CR, D), jnp.float32)),
    mesh=plsc.VectorSubcoreMesh(
        core_axis_name="c", subcore_axis_name="s",
        num_cores=NC, num_subcores=NS),
    compiler_params=pltpu.CompilerParams(needs_layout_passes=False),
    scratch_types=[
        pltpu.VMEM((EPT + C,), jnp.int32),
        pltpu.VMEM((EPT + C,), jnp.int32),
        pltpu.VMEM((TBL,), jnp.int32),
        pltpu.VMEM((2, C, D), jnp.float32),
        pltpu.VMEM((CR, D), jnp.float32),
        pltpu.VMEM((1, CR), jnp.int32),
        pltpu.VMEM((2, C), jnp.int32),
        pltpu.SemaphoreType.DMA((2,)),
        pltpu.SemaphoreType.DMA((2,)),
        pltpu.VMEM_SHARED((HN, D), jnp.float32),
        pltpu.VMEM_SHARED((CR, D), jnp.float32),
    ],
)(_sc_aggregate_body)


_GB = 1000
_GH = HALF // _GB


def _tc_q_body(ctx_r, lappe_r, z_r, peW_r, predW_r, q_r):
  W1 = predW_r[0:D, :]
  W2 = predW_r[D:D + ZD, :]
  pe_n = jnp.dot(lappe_r[...], peW_r[...], preferred_element_type=jnp.float32)
  q_r[...] = (jnp.dot(ctx_r[...], W1, preferred_element_type=jnp.float32)
              + jnp.dot(z_r[...] + pe_n, W2,
                        preferred_element_type=jnp.float32))


def _tc_loss_body(qh, cnth, lappe, tgt, z_r, peW_r, predW_r, predb_r,
                  out_r, sq_acc, pres_acc):
  i = pl.program_id(0)
  j = pl.program_id(1)

  @pl.when((i == 0) & (j == 0))
  def _():
    sq_acc[0] = 0.0
    pres_acc[0] = 0.0

  A = qh[0]
  c = cnth[...]
  present = (c > 0.0).astype(jnp.float32)
  denom = jnp.maximum(c, 1.0)
  W3 = predW_r[D + ZD:D + 2 * ZD, :]
  pe_n = jnp.dot(lappe[...], peW_r[...], preferred_element_type=jnp.float32)
  pred = (A / denom
          + jnp.dot(z_r[...] + pe_n, W3, preferred_element_type=jnp.float32)
          + predb_r[...])
  diff = (pred - tgt[...]) * present
  sq_acc[0] += jnp.sum(diff * diff)
  pres_acc[0] += jnp.sum(present)

  @pl.when((i == NC - 1) & (j == _GH - 1))
  def _():
    out_r[...] = jnp.broadcast_to(
        sq_acc[0] / (pres_acc[0] * jnp.float32(D)), (1, 1))


def kernel(lap_pe, edge_index, context_embedding, target_embedding,
           target_nodes, z, pe_W, pred_W, pred_b):
  edges = jnp.pad(edge_index.reshape(2, NS, E // NS),
                  ((0, 0), (0, 0), (0, EPT - E // NS)),
                  constant_values=-1)

  q = pl.pallas_call(
      _tc_q_body,
      grid=(N // _GB,),
      in_specs=[
          pl.BlockSpec((_GB, D), lambda i: (i, 0)),
          pl.BlockSpec((_GB, PE), lambda i: (i, 0)),
          pl.BlockSpec((1, ZD), lambda i: (0, 0)),
          pl.BlockSpec((PE, ZD), lambda i: (0, 0)),
          pl.BlockSpec((D + 2 * ZD, D), lambda i: (0, 0)),
      ],
      out_specs=pl.BlockSpec((_GB, D), lambda i: (i, 0)),
      out_shape=jax.ShapeDtypeStruct((N, D), jnp.float32),
  )(context_embedding, lap_pe, z, pe_W, pred_W)

  out_q, out_cnt = _sc_aggregate(edges, q,
                                 target_nodes.view(jnp.float32)
                                 .reshape(T // C, C),
                                 jnp.zeros((RPT, D), jnp.float32),
                                 jnp.zeros((TBL,), jnp.int32))
  cnt_col = out_cnt.reshape(NC, CR * D)[:, :HALF].reshape(N, 1)

  loss = pl.pallas_call(
      _tc_loss_body,
      grid=(NC, _GH),
      in_specs=[
          pl.BlockSpec((1, _GB, D), lambda i, j: (i, j, 0)),
          pl.BlockSpec((_GB, 1), lambda i, j: (i * _GH + j, 0)),
          pl.BlockSpec((_GB, PE), lambda i, j: (i * _GH + j, 0)),
          pl.BlockSpec((_GB, D), lambda i, j: (i * _GH + j, 0)),
          pl.BlockSpec((1, ZD), lambda i, j: (0, 0)),
          pl.BlockSpec((PE, ZD), lambda i, j: (0, 0)),
          pl.BlockSpec((D + 2 * ZD, D), lambda i, j: (0, 0)),
          pl.BlockSpec((1, D), lambda i, j: (0, 0)),
      ],
      out_specs=pl.BlockSpec((1, 1), lambda i, j: (0, 0)),
      out_shape=jax.ShapeDtypeStruct((1, 1), jnp.float32),
      scratch_shapes=[pltpu.SMEM((1,), jnp.float32),
                      pltpu.SMEM((1,), jnp.float32)],
  )(out_q, cnt_col, lap_pe, target_embedding,
    z, pe_W, pred_W, pred_b.reshape(1, D))

  return loss[0, 0]


# speedup vs baseline: 1.0064x; 1.0064x over previous
import functools

import jax
import jax.numpy as jnp
from jax import lax
from jax.experimental import pallas as pl
from jax.experimental.pallas import tpu as pltpu
from jax.experimental.pallas import tpu_sc as plsc

N = 10000
E = 320000
D = 128
ZD = 64
PE = 8
T = 1024

NC = 2
NS = 16
HALF = N // NC
C = 128
EPT = 20480
TBL = 10016
HN = 5120
RPT = HN // NS
CR = 40


def _sc_aggregate_body(edge_hbm, q_hbm, tn_hbm, zf_hbm, zi_hbm,
                       out_q, out_cnt,
                       msrc, mdst, table, q_buf, cnt_loc, iidx, sidx2,
                       gsem, ssem, acc_q, acc_cnt):
  c_idx = lax.axis_index("c")
  s_idx = lax.axis_index("s")
  lo = c_idx * HALF

  onei32 = jnp.ones((16,), jnp.int32)
  onef32 = jnp.ones((16,), jnp.float32)
  iota16 = lax.iota(jnp.int32, 16)

  iidx[0, pl.ds(0, 16)] = iota16
  iidx[0, pl.ds(16, 16)] = iota16 + 16
  iidx[0, pl.ds(24, 16)] = iota16 + 24

  base = s_idx * RPT
  pltpu.sync_copy(
      [zf_hbm, zf_hbm.at[pl.ds(0, CR)], zi_hbm],
      [acc_q.at[pl.ds(base, RPT)], cnt_loc, table])

  @pl.when(s_idx == 0)
  def _():
    pltpu.sync_copy(zf_hbm.at[pl.ds(0, CR)], acc_cnt)

  pltpu.sync_copy(tn_hbm, msrc.at[pl.ds(0, T)])
  for i in range(T // 16):
    t16 = msrc[pl.ds(i * 16, 16)]
    plsc.store_scatter(table, [t16], onei32)

  pltpu.sync_copy([edge_hbm.at[0, s_idx], edge_hbm.at[1, s_idx]],
                  [msrc.at[pl.ds(0, EPT)], mdst.at[pl.ds(0, EPT)]])

  neg1 = jnp.full((16,), -1, jnp.int32)

  def compact_one(cnt, sv, dv):
    t = plsc.load_gather(table, [jnp.maximum(sv, 0)])
    m = (t == 1) & (sv >= lo) & (sv < lo + HALF)
    rebased = jnp.where(m, sv - lo, 0)
    plsc.addupdate_scatter(
        cnt_loc,
        [lax.shift_right_logical(rebased, 7),
         jnp.bitwise_and(rebased, 127)],
        onef32, mask=m)
    plsc.store_compressed(msrc.at[pl.ds(cnt, 16)], rebased, mask=m)
    plsc.store_compressed(mdst.at[pl.ds(cnt, 16)], dv, mask=m)
    return cnt + jnp.sum(m.astype(jnp.int32))

  def compact_vec(v, cnt):
    off = v * 32
    sv0 = msrc[pl.ds(off, 16)]
    dv0 = mdst[pl.ds(off, 16)]
    sv1 = msrc[pl.ds(off + 16, 16)]
    dv1 = mdst[pl.ds(off + 16, 16)]
    cnt = compact_one(cnt, sv0, dv0)
    return compact_one(cnt, sv1, dv1)
  cnt = lax.fori_loop(0, EPT // 32, compact_vec, jnp.int32(0))

  for k in range(C // 16):
    msrc[pl.ds(cnt + k * 16, 16)] = neg1
    mdst[pl.ds(cnt + k * 16, 16)] = neg1
  nch = lax.div(cnt + C - 1, jnp.int32(C))

  plsc.subcore_barrier()

  def g_desc(ci, b):
    return pltpu.make_async_copy(
        q_hbm.at[plsc.Indices(mdst.at[pl.ds(ci * C, C)], ignored_value=-1)],
        q_buf.at[b], gsem.at[b])

  def s_desc(b):
    return pltpu.make_async_copy(
        q_buf.at[b],
        acc_q.at[plsc.Indices(sidx2.at[b], ignored_value=-1)],
        ssem.at[b])

  @pl.when(nch > 0)
  def _():
    g_desc(0, 0).start()

  def chunk(ci, _):
    b = lax.rem(ci, 2)
    nb = 1 - b

    @pl.when(ci >= 1)
    def _():
      s_desc(nb).wait()

    @pl.when(ci + 1 < nch)
    def _():
      g_desc(ci + 1, nb).start()

    g_desc(ci, b).wait()
    for j in range(C // 16):
      sidx2[b, pl.ds(j * 16, 16)] = msrc[pl.ds(ci * C + j * 16, 16)]
    s_desc(b).start(add=True)
    return 0
  lax.fori_loop(0, nch, chunk, 0)

  @pl.when(nch > 0)
  def _():
    s_desc(lax.rem(nch - 1, 2)).wait()

  pltpu.sync_copy(cnt_loc, acc_cnt.at[plsc.Indices(iidx.at[0])], add=True)

  plsc.subcore_barrier()

  pltpu.sync_copy(acc_q.at[pl.ds(base, RPT)],
                  out_q.at[c_idx, pl.ds(base, RPT)])

  @pl.when(s_idx == 0)
  def _():
    pltpu.sync_copy(acc_cnt, out_cnt.at[c_idx])


_sc_aggregate = functools.partial(
    pl.kernel,
    out_type=(jax.ShapeDtypeStruct((NC, HN, D), jnp.float32),
              jax.ShapeDtypeStruct((NC, CR, D), jnp.float32)),
    mesh=plsc.VectorSubcoreMesh(
        core_axis_name="c", subcore_axis_name="s",
        num_cores=NC, num_subcores=NS),
    compiler_params=pltpu.CompilerParams(needs_layout_passes=False),
    scratch_types=[
        pltpu.VMEM((EPT + C,), jnp.int32),
        pltpu.VMEM((EPT + C,), jnp.int32),
        pltpu.VMEM((TBL,), jnp.int32),
        pltpu.VMEM((2, C, D), jnp.float32),
        pltpu.VMEM((CR, D), jnp.float32),
        pltpu.VMEM((1, CR), jnp.int32),
        pltpu.VMEM((2, C), jnp.int32),
        pltpu.SemaphoreType.DMA((2,)),
        pltpu.SemaphoreType.DMA((2,)),
        pltpu.VMEM_SHARED((HN, D), jnp.float32),
        pltpu.VMEM_SHARED((CR, D), jnp.float32),
    ],
)(_sc_aggregate_body)


_GB = 1000
_GH = HALF // _GB


def _tc_q_body(ctx_r, lappe_r, z_r, peW_r, predW_r, q_r):
  W1 = predW_r[0:D, :]
  W2 = predW_r[D:D + ZD, :]
  pe_n = jnp.dot(lappe_r[...], peW_r[...], preferred_element_type=jnp.float32)
  q_r[...] = (jnp.dot(ctx_r[...], W1, preferred_element_type=jnp.float32)
              + jnp.dot(z_r[...] + pe_n, W2,
                        preferred_element_type=jnp.float32))


def _tc_loss_body(qh, cnth, lappe, tgt, z_r, peW_r, predW_r, predb_r,
                  out_r, sq_acc, pres_acc):
  i = pl.program_id(0)
  j = pl.program_id(1)

  @pl.when((i == 0) & (j == 0))
  def _():
    sq_acc[0] = 0.0
    pres_acc[0] = 0.0

  A = qh[0]
  c = cnth[...]
  present = (c > 0.0).astype(jnp.float32)
  denom = jnp.maximum(c, 1.0)
  W3 = predW_r[D + ZD:D + 2 * ZD, :]
  pe_n = jnp.dot(lappe[...], peW_r[...], preferred_element_type=jnp.float32)
  pred = (A / denom
          + jnp.dot(z_r[...] + pe_n, W3, preferred_element_type=jnp.float32)
          + predb_r[...])
  diff = (pred - tgt[...]) * present
  sq_acc[0] += jnp.sum(diff * diff)
  pres_acc[0] += jnp.sum(present)

  @pl.when((i == NC - 1) & (j == _GH - 1))
  def _():
    out_r[...] = jnp.broadcast_to(
        sq_acc[0] / (pres_acc[0] * jnp.float32(D)), (1, 1))


def kernel(lap_pe, edge_index, context_embedding, target_embedding,
           target_nodes, z, pe_W, pred_W, pred_b):
  edges = jnp.pad(edge_index.reshape(2, NS, E // NS),
                  ((0, 0), (0, 0), (0, EPT - E // NS)),
                  constant_values=-1)

  q = pl.pallas_call(
      _tc_q_body,
      grid=(N // _GB,),
      in_specs=[
          pl.BlockSpec((_GB, D), lambda i: (i, 0)),
          pl.BlockSpec((_GB, PE), lambda i: (i, 0)),
          pl.BlockSpec((1, ZD), lambda i: (0, 0)),
          pl.BlockSpec((PE, ZD), lambda i: (0, 0)),
          pl.BlockSpec((D + 2 * ZD, D), lambda i: (0, 0)),
      ],
      out_specs=pl.BlockSpec((_GB, D), lambda i: (i, 0)),
      out_shape=jax.ShapeDtypeStruct((N, D), jnp.float32),
  )(context_embedding, lap_pe, z, pe_W, pred_W)

  out_q, out_cnt = _sc_aggregate(edges, q, target_nodes,
                                 jnp.zeros((RPT, D), jnp.float32),
                                 jnp.zeros((TBL,), jnp.int32))
  cnt_col = out_cnt.reshape(NC, CR * D)[:, :HALF].reshape(N, 1)

  loss = pl.pallas_call(
      _tc_loss_body,
      grid=(NC, _GH),
      in_specs=[
          pl.BlockSpec((1, _GB, D), lambda i, j: (i, j, 0)),
          pl.BlockSpec((_GB, 1), lambda i, j: (i * _GH + j, 0)),
          pl.BlockSpec((_GB, PE), lambda i, j: (i * _GH + j, 0)),
          pl.BlockSpec((_GB, D), lambda i, j: (i * _GH + j, 0)),
          pl.BlockSpec((1, ZD), lambda i, j: (0, 0)),
          pl.BlockSpec((PE, ZD), lambda i, j: (0, 0)),
          pl.BlockSpec((D + 2 * ZD, D), lambda i, j: (0, 0)),
          pl.BlockSpec((1, D), lambda i, j: (0, 0)),
      ],
      out_specs=pl.BlockSpec((1, 1), lambda i, j: (0, 0)),
      out_shape=jax.ShapeDtypeStruct((1, 1), jnp.float32),
      scratch_shapes=[pltpu.SMEM((1,), jnp.float32),
                      pltpu.SMEM((1,), jnp.float32)],
  )(out_q, cnt_col, lap_pe, target_embedding,
    z, pe_W, pred_W, pred_b.reshape(1, D))

  return loss[0, 0]
